# baseline (speedup 1.0000x reference)
import jax
import jax.numpy as jnp
from jax import lax
from jax.experimental import pallas as pl
from jax.experimental.pallas import tpu as pltpu
from jax.experimental.pallas import tpu_sc as plsc

EPS = 1e-5
HEADS = 4
OUT_F = 32
D = HEADS * OUT_F
N_NODES = 10000
E_TOTAL = 320000
NW = 32
E_PER_W = E_TOTAL // NW
CHUNK = 80
N_CHUNKS = E_PER_W // CHUNK
GROUPS = CHUNK // 16
PCOLS = 16
NCH_TOTAL = N_NODES // CHUNK

_LN2 = 0.6931471805599453


def _rsqrt(a):
    a_ = jnp.maximum(a, 1e-30)
    i = lax.bitcast_convert_type(a_, jnp.int32)
    i = jnp.int32(0x5F3759DF) - lax.shift_right_logical(i, 1)
    y = lax.bitcast_convert_type(i, jnp.float32)
    for _ in range(2):
        y = y * (1.5 - 0.5 * a_ * y * y)
    return y


_GDN = lax.GatherDimensionNumbers(
    offset_dims=(), collapsed_slice_dims=(0,), start_index_map=(0,))


def _lperm(v, idx):
    return lax.gather(v, idx[:, None], _GDN, (1,),
                      mode=lax.GatherScatterMode.PROMISE_IN_BOUNDS)


def _sqrt(a):
    return a * _rsqrt(a)


def _log(z):
    i = lax.bitcast_convert_type(z, jnp.int32)
    e = lax.shift_right_logical(i, 23) - 127
    m = lax.bitcast_convert_type(
        jnp.bitwise_or(jnp.bitwise_and(i, jnp.int32(0x007FFFFF)),
                       jnp.int32(0x3F800000)), jnp.float32)
    s = (m - 1.0) / (m + 1.0)
    s2 = s * s
    p = 2.0 + s2 * (2.0 / 3.0 + s2 * (2.0 / 5.0 + s2 * (2.0 / 7.0 + s2 * (2.0 / 9.0))))
    return e.astype(jnp.float32) * _LN2 + s * p


def _atanh(x):
    return 0.5 * _log((1.0 + x) / (1.0 - x))


def _tanh(x):
    t = jnp.exp(2.0 * x)
    return 1.0 - 2.0 / (t + 1.0)


def _node_body(x_hbm, abt_hbm, p_hbm, abt_v, rows_v, pstage, sem):
    wid = lax.axis_index("s") * 2 + lax.axis_index("c")
    pltpu.sync_copy(abt_hbm, abt_v)
    iota16 = lax.iota(jnp.int32, 16)
    lo = wid * NCH_TOTAL // NW
    hi = (wid + 1) * NCH_TOTAL // NW

    @pl.loop(lo, hi)
    def _nchunk(c):
        nrow = pl.multiple_of(c * CHUNK, 8)
        pltpu.async_copy(x_hbm.at[pl.ds(nrow, CHUNK)], rows_v, sem).wait()

        @pl.loop(0, GROUPS)
        def _ngroup(g):
            rows16 = iota16 + g * 16
            acc_a1 = [None] * HEADS
            acc_a2 = [None] * HEADS
            acc_n2 = [None] * HEADS
            for k in range(D):
                h = k // OUT_F
                kvec = jnp.full((16,), k, jnp.int32)
                xk = plsc.load_gather(rows_v, [rows16, kvec])
                c1 = xk * abt_v[0, k, :]
                c2 = xk * abt_v[1, k, :]
                c3 = xk * xk
                if acc_a1[h] is None:
                    acc_a1[h], acc_a2[h], acc_n2[h] = c1, c2, c3
                else:
                    acc_a1[h] = acc_a1[h] + c1
                    acc_a2[h] = acc_a2[h] + c2
                    acc_n2[h] = acc_n2[h] + c3
            sbase = g * (16 * PCOLS) + iota16 * PCOLS
            for h in range(HEADS):
                plsc.store_scatter(pstage, [sbase + h], acc_a1[h])
                plsc.store_scatter(pstage, [sbase + (4 + h)], acc_a2[h])
                plsc.store_scatter(pstage, [sbase + (8 + h)], acc_n2[h])

        pbase = pl.multiple_of(nrow * PCOLS, 8)
        pltpu.sync_copy(pstage, p_hbm.at[pl.ds(pbase, CHUNK * PCOLS)])


def _edge_body(x_hbm, p_hbm, src_hbm, dst_hbm, bpt_hbm, out_hbm,
               bpt_v, idx_src, idx_dst, rows0, pi0, pj0, rows1, pi1, pj1,
               out_stage, sem0, sem1):
    wid = lax.axis_index("s") * 2 + lax.axis_index("c")
    pltpu.sync_copy(bpt_hbm, bpt_v)
    iota16 = lax.iota(jnp.int32, 16)
    perm1 = jnp.bitwise_xor(iota16, 1)
    perm2 = jnp.bitwise_xor(iota16, 2)
    ebase0 = wid * E_PER_W

    def _issue(c, rows_v, pi_v, pj_v, sem):
        ebase = pl.multiple_of(ebase0 + c * CHUNK, 8)
        pltpu.sync_copy(src_hbm.at[pl.ds(ebase, CHUNK)], idx_src)
        pltpu.sync_copy(dst_hbm.at[pl.ds(ebase, CHUNK)], idx_dst)
        pltpu.async_copy(x_hbm.at[idx_src], rows_v, sem)
        pltpu.async_copy(p_hbm.at[idx_src], pj_v, sem)
        pltpu.async_copy(p_hbm.at[idx_dst], pi_v, sem)

    def _drain(rows_v, pi_v, pj_v, sem):
        pltpu.make_async_copy(x_hbm.at[pl.ds(0, CHUNK)], rows_v, sem).wait()
        pltpu.make_async_copy(p_hbm.at[pl.ds(0, CHUNK)], pj_v, sem).wait()
        pltpu.make_async_copy(p_hbm.at[pl.ds(0, CHUNK)], pi_v, sem).wait()

    def _compute(c, rows_v, pi_v, pj_v):
        @pl.loop(0, GROUPS, unroll=2)
        def _group(g):
            rows16 = iota16 + g * 16

            def pick(ref, col):
                cvec = jnp.full((16,), col, jnp.int32)
                return plsc.load_gather(ref, [rows16, cvec])

            alpha = [pick(pi_v, h) + pick(pj_v, 4 + h) for h in range(HEADS)]
            nv2 = [pick(pj_v, 8 + h) for h in range(HEADS)]

            n2 = (alpha[0] * alpha[0] + alpha[1] * alpha[1]) + \
                 (alpha[2] * alpha[2] + alpha[3] * alpha[3])
            n = _sqrt(n2)
            nc = jnp.minimum(jnp.maximum(n, 1e-15), 1.0 - EPS)
            scale = _atanh(nc) / jnp.maximum(n, 1e-15)
            ainf = [alpha[h] * scale for h in range(HEADS)]

            mx = jnp.maximum(jnp.maximum(ainf[0], ainf[1]),
                             jnp.maximum(ainf[2], ainf[3]))
            ex = [jnp.exp(ainf[h] - mx) for h in range(HEADS)]
            rs = 1.0 / ((ex[0] + ex[1]) + (ex[2] + ex[3]))
            sm = [ex[h] * rs for h in range(HEADS)]

            n2p = (sm[0] * sm[0] + sm[1] * sm[1]) + \
                  (sm[2] * sm[2] + sm[3] * sm[3])
            npv = _sqrt(n2p)
            fac = _tanh(npv) / jnp.maximum(npv, 1e-15)

            coef = [None] * HEADS
            for h in range(HEADS):
                nv = _sqrt(nv2[h])
                ncv = jnp.minimum(jnp.maximum(nv, 1e-15), 1.0 - EPS)
                t = (sm[h] * fac) * _atanh(ncv)
                coef[h] = _tanh(t) / jnp.maximum(nv, 1e-15)

            def xjg(f):
                fvec = jnp.full((16,), f, jnp.int32)
                return plsc.load_gather(rows_v, [rows16, fvec])

            out = [coef[0] * xjg(k) for k in range(OUT_F)]
            x2 = nv2[0] * coef[0] * coef[0]
            for h in range(1, HEADS):
                y2 = nv2[h] * coef[h] * coef[h]
                xyp = [out[k] * xjg(32 * h + k) for k in range(4)]
                for k in range(4, OUT_F):
                    xyp[k % 4] = xyp[k % 4] + out[k] * xjg(32 * h + k)
                xy = ((xyp[0] + xyp[1]) + (xyp[2] + xyp[3])) * coef[h]
                cx = 1.0 + 2.0 * xy + y2
                cyb = 1.0 - x2
                cy = cyb * coef[h]
                den = 1.0 + 2.0 * xy + x2 * y2
                rden = 1.0 / jnp.maximum(den, 1e-15)
                cxr = cx * rden
                cyr = cy * rden
                for k in range(OUT_F):
                    out[k] = cxr * out[k] + cyr * xjg(32 * h + k)
                x2 = (cx * cx * x2 + 2.0 * cx * cyb * xy
                      + cyb * cyb * y2) * (rden * rden)

            def _group4(v):
                v1 = v + _lperm(v, perm1)
                return v1 + _lperm(v1, perm2)

            b2 = bpt_v[32, :]
            xyp = [out[k] * bpt_v[k, :] for k in range(4)]
            for k in range(4, OUT_F):
                xyp[k % 4] = xyp[k % 4] + out[k] * bpt_v[k, :]
            xy = _group4((xyp[0] + xyp[1]) + (xyp[2] + xyp[3]))
            x2g = _group4(x2)
            cx = 1.0 + 2.0 * xy + b2
            cy = 1.0 - x2g
            den = 1.0 + 2.0 * xy + x2g * b2
            rden = 1.0 / jnp.maximum(den, 1e-15)
            cxr = cx * rden
            cyr = cy * rden
            rbase = lax.shift_left(rows16, 5)
            for k in range(OUT_F):
                v = cxr * out[k] + cyr * bpt_v[k, :]
                plsc.store_scatter(out_stage, [rbase + k], v)

        obase = pl.multiple_of((ebase0 + c * CHUNK) * OUT_F, 8)
        pltpu.sync_copy(out_stage, out_hbm.at[pl.ds(obase, CHUNK * OUT_F)])

    _issue(0, rows0, pi0, pj0, sem0)

    @pl.loop(0, N_CHUNKS)
    def _chunk(c):
        nxt = jnp.minimum(c + 1, N_CHUNKS - 1)
        even = jnp.bitwise_and(c, 1) == 0

        @pl.when(even)
        def _():
            _drain(rows0, pi0, pj0, sem0)
            _issue(nxt, rows1, pi1, pj1, sem1)
            _compute(c, rows0, pi0, pj0)

        @pl.when(jnp.logical_not(even))
        def _():
            _drain(rows1, pi1, pj1, sem1)
            _issue(nxt, rows0, pi0, pj0, sem0)
            _compute(c, rows1, pi1, pj1)

    _drain(rows1, pi1, pj1, sem1)


@jax.jit
def kernel(x, edge_index, att, bias):
    def _expmap0(u):
        n = jnp.maximum(jnp.sqrt(jnp.sum(u * u, axis=-1, keepdims=True)), 1e-15)
        return jnp.tanh(n) * u / n

    att_h = _expmap0(att).reshape(HEADS, 2 * OUT_F)
    a1 = att_h[:, :OUT_F].reshape(-1)
    a2 = att_h[:, OUT_F:].reshape(-1)
    abt = jnp.broadcast_to(
        jnp.stack([a1, a2])[:, :, None], (2, D, 16)).astype(jnp.float32)
    bh = _expmap0(bias)
    lanemod = jnp.arange(16) % 4
    b_pat = bh[32 * lanemod[None, :] + jnp.arange(32)[:, None]]
    b2 = jnp.broadcast_to(jnp.sum(bh * bh), (1, 16))
    bpt = jnp.concatenate([b_pat, b2], axis=0).astype(jnp.float32)

    mesh = plsc.VectorSubcoreMesh(core_axis_name="c", subcore_axis_name="s")
    cparams = pltpu.CompilerParams(needs_layout_passes=False,
                                   use_tc_tiling_on_sc=False)

    node_run = pl.kernel(
        _node_body,
        out_type=jax.ShapeDtypeStruct((N_NODES * PCOLS,), jnp.float32),
        mesh=mesh,
        compiler_params=cparams,
        scratch_types=[
            pltpu.VMEM((2, D, 16), jnp.float32),
            pltpu.VMEM((CHUNK, D), jnp.float32),
            pltpu.VMEM((CHUNK * PCOLS,), jnp.float32),
            pltpu.SemaphoreType.DMA,
        ],
    )
    p_tab = node_run(x, abt).reshape(N_NODES, PCOLS)

    edge_run = pl.kernel(
        _edge_body,
        out_type=jax.ShapeDtypeStruct((E_TOTAL * OUT_F,), jnp.float32),
        mesh=mesh,
        compiler_params=cparams,
        scratch_types=[
            pltpu.VMEM((33, 16), jnp.float32),
            pltpu.VMEM((CHUNK,), jnp.int32),
            pltpu.VMEM((CHUNK,), jnp.int32),
            pltpu.VMEM((CHUNK, D), jnp.float32),
            pltpu.VMEM((CHUNK, PCOLS), jnp.float32),
            pltpu.VMEM((CHUNK, PCOLS), jnp.float32),
            pltpu.VMEM((CHUNK, D), jnp.float32),
            pltpu.VMEM((CHUNK, PCOLS), jnp.float32),
            pltpu.VMEM((CHUNK, PCOLS), jnp.float32),
            pltpu.VMEM((CHUNK * OUT_F,), jnp.float32),
            pltpu.SemaphoreType.DMA,
            pltpu.SemaphoreType.DMA,
        ],
    )
    out = edge_run(x, p_tab, edge_index[0], edge_index[1], bpt)
    return out.reshape(E_TOTAL // 4, D)


# speedup vs baseline: 1.2396x; 1.2396x over previous
import jax
import jax.numpy as jnp
from jax import lax
from jax.experimental import pallas as pl
from jax.experimental.pallas import tpu as pltpu
from jax.experimental.pallas import tpu_sc as plsc

EPS = 1e-5
HEADS = 4
OUT_F = 32
D = HEADS * OUT_F
N_NODES = 10000
E_TOTAL = 320000
NW = 32
E_PER_W = E_TOTAL // NW
CHUNK = 80
N_CHUNKS = E_PER_W // CHUNK
GROUPS = CHUNK // 16
PCOLS = 16
NCH_TOTAL = N_NODES // CHUNK

_LN2 = 0.6931471805599453


def _rsqrt(a):
    a_ = jnp.maximum(a, 1e-30)
    i = lax.bitcast_convert_type(a_, jnp.int32)
    i = jnp.int32(0x5F3759DF) - lax.shift_right_logical(i, 1)
    y = lax.bitcast_convert_type(i, jnp.float32)
    for _ in range(2):
        y = y * (1.5 - 0.5 * a_ * y * y)
    return y


_GDN = lax.GatherDimensionNumbers(
    offset_dims=(), collapsed_slice_dims=(0,), start_index_map=(0,))


def _lperm(v, idx):
    return lax.gather(v, idx[:, None], _GDN, (1,),
                      mode=lax.GatherScatterMode.PROMISE_IN_BOUNDS)


def _sqrt(a):
    return a * _rsqrt(a)


def _log(z):
    i = lax.bitcast_convert_type(z, jnp.int32)
    e = lax.shift_right_logical(i, 23) - 127
    m = lax.bitcast_convert_type(
        jnp.bitwise_or(jnp.bitwise_and(i, jnp.int32(0x007FFFFF)),
                       jnp.int32(0x3F800000)), jnp.float32)
    s = (m - 1.0) / (m + 1.0)
    s2 = s * s
    p = 2.0 + s2 * (2.0 / 3.0 + s2 * (2.0 / 5.0 + s2 * (2.0 / 7.0 + s2 * (2.0 / 9.0))))
    return e.astype(jnp.float32) * _LN2 + s * p


def _atanh(x):
    return 0.5 * _log((1.0 + x) / (1.0 - x))


def _tanh(x):
    t = jnp.exp(2.0 * x)
    return 1.0 - 2.0 / (t + 1.0)


def _node_body(x_hbm, abt_hbm, p_hbm, abt_v, rows_v, pstage, sem):
    wid = lax.axis_index("s") * 2 + lax.axis_index("c")
    pltpu.sync_copy(abt_hbm, abt_v)
    iota16 = lax.iota(jnp.int32, 16)
    lo = wid * NCH_TOTAL // NW
    hi = (wid + 1) * NCH_TOTAL // NW

    @pl.loop(lo, hi)
    def _nchunk(c):
        nrow = pl.multiple_of(c * CHUNK, 8)
        pltpu.async_copy(x_hbm.at[pl.ds(nrow, CHUNK)], rows_v, sem).wait()

        @pl.loop(0, GROUPS)
        def _ngroup(g):
            rows16 = iota16 + g * 16
            acc_a1 = [None] * HEADS
            acc_a2 = [None] * HEADS
            acc_n2 = [None] * HEADS
            for k in range(D):
                h = k // OUT_F
                kvec = jnp.full((16,), k, jnp.int32)
                xk = plsc.load_gather(rows_v, [rows16, kvec])
                c1 = xk * abt_v[0, k, :]
                c2 = xk * abt_v[1, k, :]
                c3 = xk * xk
                if acc_a1[h] is None:
                    acc_a1[h], acc_a2[h], acc_n2[h] = c1, c2, c3
                else:
                    acc_a1[h] = acc_a1[h] + c1
                    acc_a2[h] = acc_a2[h] + c2
                    acc_n2[h] = acc_n2[h] + c3
            sbase = g * (16 * PCOLS) + iota16 * PCOLS
            for h in range(HEADS):
                plsc.store_scatter(pstage, [sbase + h], acc_a1[h])
                plsc.store_scatter(pstage, [sbase + (4 + h)], acc_a2[h])
                plsc.store_scatter(pstage, [sbase + (8 + h)], acc_n2[h])

        pbase = pl.multiple_of(nrow * PCOLS, 8)
        pltpu.sync_copy(pstage, p_hbm.at[pl.ds(pbase, CHUNK * PCOLS)])


def _edge_body(x_hbm, p_hbm, src_hbm, dst_hbm, bpt_hbm, out_hbm,
               bpt_v, idx_src, idx_dst, rows0, pi0, pj0, rows1, pi1, pj1,
               out_stage, sem0, sem1):
    wid = lax.axis_index("s") * 2 + lax.axis_index("c")
    pltpu.sync_copy(bpt_hbm, bpt_v)
    iota16 = lax.iota(jnp.int32, 16)
    perm1 = jnp.bitwise_xor(iota16, 1)
    perm2 = jnp.bitwise_xor(iota16, 2)
    ebase0 = wid * E_PER_W

    def _issue(c, rows_v, pi_v, pj_v, sem):
        ebase = pl.multiple_of(ebase0 + c * CHUNK, 8)
        pltpu.sync_copy(src_hbm.at[pl.ds(ebase, CHUNK)], idx_src)
        pltpu.sync_copy(dst_hbm.at[pl.ds(ebase, CHUNK)], idx_dst)
        pltpu.async_copy(x_hbm.at[idx_src], rows_v, sem)
        pltpu.async_copy(p_hbm.at[idx_src], pj_v, sem)
        pltpu.async_copy(p_hbm.at[idx_dst], pi_v, sem)

    def _drain(rows_v, pi_v, pj_v, sem):
        pltpu.make_async_copy(x_hbm.at[pl.ds(0, CHUNK)], rows_v, sem).wait()
        pltpu.make_async_copy(p_hbm.at[pl.ds(0, CHUNK)], pj_v, sem).wait()
        pltpu.make_async_copy(p_hbm.at[pl.ds(0, CHUNK)], pi_v, sem).wait()

    def _compute(c, rows_v, pi_v, pj_v):
        @pl.loop(0, GROUPS)
        def _group(g):
            rows16 = iota16 + g * 16

            def pick(ref, col):
                cvec = jnp.full((16,), col, jnp.int32)
                return plsc.load_gather(ref, [rows16, cvec])

            alpha = [pick(pi_v, h) + pick(pj_v, 4 + h) for h in range(HEADS)]
            nv2 = [pick(pj_v, 8 + h) for h in range(HEADS)]

            n2 = (alpha[0] * alpha[0] + alpha[1] * alpha[1]) + \
                 (alpha[2] * alpha[2] + alpha[3] * alpha[3])
            n = _sqrt(n2)
            nc = jnp.minimum(jnp.maximum(n, 1e-15), 1.0 - EPS)
            scale = _atanh(nc) / jnp.maximum(n, 1e-15)
            ainf = [alpha[h] * scale for h in range(HEADS)]

            mx = jnp.maximum(jnp.maximum(ainf[0], ainf[1]),
                             jnp.maximum(ainf[2], ainf[3]))
            ex = [jnp.exp(ainf[h] - mx) for h in range(HEADS)]
            rs = 1.0 / ((ex[0] + ex[1]) + (ex[2] + ex[3]))
            sm = [ex[h] * rs for h in range(HEADS)]

            n2p = (sm[0] * sm[0] + sm[1] * sm[1]) + \
                  (sm[2] * sm[2] + sm[3] * sm[3])
            npv = _sqrt(n2p)
            fac = _tanh(npv) / jnp.maximum(npv, 1e-15)

            coef = [None] * HEADS
            for h in range(HEADS):
                nv = _sqrt(nv2[h])
                ncv = jnp.minimum(jnp.maximum(nv, 1e-15), 1.0 - EPS)
                t = (sm[h] * fac) * _atanh(ncv)
                coef[h] = _tanh(t) / jnp.maximum(nv, 1e-15)

            def xjg(f):
                fvec = jnp.full((16,), f, jnp.int32)
                return plsc.load_gather(rows_v, [rows16, fvec])

            out = [coef[0] * xjg(k) for k in range(OUT_F)]
            x2 = nv2[0] * coef[0] * coef[0]
            for h in range(1, HEADS):
                y2 = nv2[h] * coef[h] * coef[h]
                xyp = [out[k] * xjg(32 * h + k) for k in range(4)]
                for k in range(4, OUT_F):
                    xyp[k % 4] = xyp[k % 4] + out[k] * xjg(32 * h + k)
                xy = ((xyp[0] + xyp[1]) + (xyp[2] + xyp[3])) * coef[h]
                cx = 1.0 + 2.0 * xy + y2
                cyb = 1.0 - x2
                cy = cyb * coef[h]
                den = 1.0 + 2.0 * xy + x2 * y2
                rden = 1.0 / jnp.maximum(den, 1e-15)
                cxr = cx * rden
                cyr = cy * rden
                for k in range(OUT_F):
                    out[k] = cxr * out[k] + cyr * xjg(32 * h + k)
                x2 = (cx * cx * x2 + 2.0 * cx * cyb * xy
                      + cyb * cyb * y2) * (rden * rden)

            def _group4(v):
                v1 = v + _lperm(v, perm1)
                return v1 + _lperm(v1, perm2)

            b2 = bpt_v[32, :]
            xyp = [out[k] * bpt_v[k, :] for k in range(4)]
            for k in range(4, OUT_F):
                xyp[k % 4] = xyp[k % 4] + out[k] * bpt_v[k, :]
            xy = _group4((xyp[0] + xyp[1]) + (xyp[2] + xyp[3]))
            x2g = _group4(x2)
            cx = 1.0 + 2.0 * xy + b2
            cy = 1.0 - x2g
            den = 1.0 + 2.0 * xy + x2g * b2
            rden = 1.0 / jnp.maximum(den, 1e-15)
            cxr = cx * rden
            cyr = cy * rden
            rbase = lax.shift_left(rows16, 5)
            for k in range(OUT_F):
                v = cxr * out[k] + cyr * bpt_v[k, :]
                plsc.store_scatter(out_stage, [rbase + k], v)

        obase = pl.multiple_of((ebase0 + c * CHUNK) * OUT_F, 8)
        pltpu.sync_copy(out_stage, out_hbm.at[pl.ds(obase, CHUNK * OUT_F)])

    _issue(0, rows0, pi0, pj0, sem0)

    @pl.loop(0, N_CHUNKS)
    def _chunk(c):
        nxt = jnp.minimum(c + 1, N_CHUNKS - 1)
        even = jnp.bitwise_and(c, 1) == 0

        @pl.when(even)
        def _():
            _drain(rows0, pi0, pj0, sem0)
            _issue(nxt, rows1, pi1, pj1, sem1)
            _compute(c, rows0, pi0, pj0)

        @pl.when(jnp.logical_not(even))
        def _():
            _drain(rows1, pi1, pj1, sem1)
            _issue(nxt, rows0, pi0, pj0, sem0)
            _compute(c, rows1, pi1, pj1)

    _drain(rows1, pi1, pj1, sem1)


@jax.jit
def kernel(x, edge_index, att, bias):
    def _expmap0(u):
        n = jnp.maximum(jnp.sqrt(jnp.sum(u * u, axis=-1, keepdims=True)), 1e-15)
        return jnp.tanh(n) * u / n

    att_h = _expmap0(att).reshape(HEADS, 2 * OUT_F)
    a1 = att_h[:, :OUT_F].reshape(-1)
    a2 = att_h[:, OUT_F:].reshape(-1)
    abt = jnp.broadcast_to(
        jnp.stack([a1, a2])[:, :, None], (2, D, 16)).astype(jnp.float32)
    bh = _expmap0(bias)
    lanemod = jnp.arange(16) % 4
    b_pat = bh[32 * lanemod[None, :] + jnp.arange(32)[:, None]]
    b2 = jnp.broadcast_to(jnp.sum(bh * bh), (1, 16))
    bpt = jnp.concatenate([b_pat, b2], axis=0).astype(jnp.float32)

    mesh = plsc.VectorSubcoreMesh(core_axis_name="c", subcore_axis_name="s")
    cparams = pltpu.CompilerParams(needs_layout_passes=False,
                                   use_tc_tiling_on_sc=False)

    node_run = pl.kernel(
        _node_body,
        out_type=jax.ShapeDtypeStruct((N_NODES * PCOLS,), jnp.float32),
        mesh=mesh,
        compiler_params=cparams,
        scratch_types=[
            pltpu.VMEM((2, D, 16), jnp.float32),
            pltpu.VMEM((CHUNK, D), jnp.float32),
            pltpu.VMEM((CHUNK * PCOLS,), jnp.float32),
            pltpu.SemaphoreType.DMA,
        ],
    )
    p_tab = node_run(x, abt).reshape(N_NODES, PCOLS)

    edge_run = pl.kernel(
        _edge_body,
        out_type=jax.ShapeDtypeStruct((E_TOTAL * OUT_F,), jnp.float32),
        mesh=mesh,
        compiler_params=cparams,
        scratch_types=[
            pltpu.VMEM((33, 16), jnp.float32),
            pltpu.VMEM((CHUNK,), jnp.int32),
            pltpu.VMEM((CHUNK,), jnp.int32),
            pltpu.VMEM((CHUNK, D), jnp.float32),
            pltpu.VMEM((CHUNK, PCOLS), jnp.float32),
            pltpu.VMEM((CHUNK, PCOLS), jnp.float32),
            pltpu.VMEM((CHUNK, D), jnp.float32),
            pltpu.VMEM((CHUNK, PCOLS), jnp.float32),
            pltpu.VMEM((CHUNK, PCOLS), jnp.float32),
            pltpu.VMEM((CHUNK * OUT_F,), jnp.float32),
            pltpu.SemaphoreType.DMA,
            pltpu.SemaphoreType.DMA,
        ],
    )
    out = edge_run(x, p_tab, edge_index[0], edge_index[1], bpt)
    return out.reshape(E_TOTAL // 4, D)


# speedup vs baseline: 1.3856x; 1.1177x over previous
import jax
import jax.numpy as jnp
from jax import lax
from jax.experimental import pallas as pl
from jax.experimental.pallas import tpu as pltpu
from jax.experimental.pallas import tpu_sc as plsc

EPS = 1e-5
HEADS = 4
OUT_F = 32
D = HEADS * OUT_F
N_NODES = 10000
E_TOTAL = 320000
NW = 32
E_PER_W = E_TOTAL // NW
CHUNK = 80
N_CHUNKS = E_PER_W // CHUNK
GROUPS = CHUNK // 16
PCOLS = 16
NCH_TOTAL = N_NODES // CHUNK

_LN2 = 0.6931471805599453


def _rsqrt(a):
    a_ = jnp.maximum(a, 1e-30)
    i = lax.bitcast_convert_type(a_, jnp.int32)
    i = jnp.int32(0x5F3759DF) - lax.shift_right_logical(i, 1)
    y = lax.bitcast_convert_type(i, jnp.float32)
    for _ in range(2):
        y = y * (1.5 - 0.5 * a_ * y * y)
    return y


_GDN = lax.GatherDimensionNumbers(
    offset_dims=(), collapsed_slice_dims=(0,), start_index_map=(0,))


def _lperm(v, idx):
    return lax.gather(v, idx[:, None], _GDN, (1,),
                      mode=lax.GatherScatterMode.PROMISE_IN_BOUNDS)


def _sqrt(a):
    return a * _rsqrt(a)


def _log(z):
    i = lax.bitcast_convert_type(z, jnp.int32)
    e = lax.shift_right_logical(i, 23) - 127
    m = lax.bitcast_convert_type(
        jnp.bitwise_or(jnp.bitwise_and(i, jnp.int32(0x007FFFFF)),
                       jnp.int32(0x3F800000)), jnp.float32)
    s = (m - 1.0) / (m + 1.0)
    s2 = s * s
    p = 2.0 + s2 * (2.0 / 3.0 + s2 * (2.0 / 5.0 + s2 * (2.0 / 7.0 + s2 * (2.0 / 9.0))))
    return e.astype(jnp.float32) * _LN2 + s * p


def _atanh(x):
    return 0.5 * _log((1.0 + x) / (1.0 - x))


def _tanh(x):
    t = jnp.exp(2.0 * x)
    return 1.0 - 2.0 / (t + 1.0)


def _node_body(x_hbm, abt_hbm, p_hbm, abt_v, rows_v, pstage, sem):
    wid = lax.axis_index("s") * 2 + lax.axis_index("c")
    pltpu.sync_copy(abt_hbm, abt_v)
    iota16 = lax.iota(jnp.int32, 16)
    lo = wid * NCH_TOTAL // NW
    hi = (wid + 1) * NCH_TOTAL // NW

    @pl.loop(lo, hi)
    def _nchunk(c):
        nrow = pl.multiple_of(c * CHUNK, 8)
        pltpu.async_copy(x_hbm.at[pl.ds(nrow, CHUNK)], rows_v, sem).wait()

        @pl.loop(0, GROUPS)
        def _ngroup(g):
            rows16 = iota16 + g * 16
            acc_a1 = [None] * HEADS
            acc_a2 = [None] * HEADS
            acc_n2 = [None] * HEADS
            for k in range(D):
                h = k // OUT_F
                kvec = jnp.full((16,), k, jnp.int32)
                xk = plsc.load_gather(rows_v, [rows16, kvec])
                c1 = xk * abt_v[0, k, :]
                c2 = xk * abt_v[1, k, :]
                c3 = xk * xk
                if acc_a1[h] is None:
                    acc_a1[h], acc_a2[h], acc_n2[h] = c1, c2, c3
                else:
                    acc_a1[h] = acc_a1[h] + c1
                    acc_a2[h] = acc_a2[h] + c2
                    acc_n2[h] = acc_n2[h] + c3
            sbase = g * (16 * PCOLS) + iota16 * PCOLS
            for h in range(HEADS):
                plsc.store_scatter(pstage, [sbase + h], acc_a1[h])
                plsc.store_scatter(pstage, [sbase + (4 + h)], acc_a2[h])
                plsc.store_scatter(pstage, [sbase + (8 + h)], acc_n2[h])

        pbase = pl.multiple_of(nrow * PCOLS, 8)
        pltpu.sync_copy(pstage, p_hbm.at[pl.ds(pbase, CHUNK * PCOLS)])


def _edge_body(x_hbm, p_hbm, src_hbm, dst_hbm, bpt_hbm, out_hbm,
               bpt_v, is0, id0, is1, id1, rows0, pi0, pj0, rows1, pi1, pj1,
               out0, out1, si0, si1, sg0, sg1, so0, so1):
    wid = lax.axis_index("s") * 2 + lax.axis_index("c")
    pltpu.sync_copy(bpt_hbm, bpt_v)
    iota16 = lax.iota(jnp.int32, 16)
    perm1 = jnp.bitwise_xor(iota16, 1)
    perm2 = jnp.bitwise_xor(iota16, 2)
    ebase0 = wid * E_PER_W

    def _issue_idx(c, i_s, i_d, sem):
        ebase = pl.multiple_of(ebase0 + c * CHUNK, 8)
        pltpu.async_copy(src_hbm.at[pl.ds(ebase, CHUNK)], i_s, sem)
        pltpu.async_copy(dst_hbm.at[pl.ds(ebase, CHUNK)], i_d, sem)

    def _drain_idx(i_s, i_d, sem):
        pltpu.make_async_copy(src_hbm.at[pl.ds(0, CHUNK)], i_s, sem).wait()
        pltpu.make_async_copy(src_hbm.at[pl.ds(0, CHUNK)], i_d, sem).wait()

    def _issue_g(i_s, i_d, rows_v, pi_v, pj_v, sem):
        pltpu.async_copy(x_hbm.at[i_s], rows_v, sem)
        pltpu.async_copy(p_hbm.at[i_s], pj_v, sem)
        pltpu.async_copy(p_hbm.at[i_d], pi_v, sem)

    def _drain_g(rows_v, pi_v, pj_v, sem):
        pltpu.make_async_copy(x_hbm.at[pl.ds(0, CHUNK)], rows_v, sem).wait()
        pltpu.make_async_copy(p_hbm.at[pl.ds(0, CHUNK)], pj_v, sem).wait()
        pltpu.make_async_copy(p_hbm.at[pl.ds(0, CHUNK)], pi_v, sem).wait()

    def _drain_out(out_stage, sem):
        pltpu.make_async_copy(
            out_stage, out_hbm.at[pl.ds(0, CHUNK * OUT_F)], sem).wait()

    def _compute(c, rows_v, pi_v, pj_v, out_stage, semo):
        @pl.loop(0, GROUPS)
        def _group(g):
            rows16 = iota16 + g * 16

            def pick(ref, col):
                cvec = jnp.full((16,), col, jnp.int32)
                return plsc.load_gather(ref, [rows16, cvec])

            alpha = [pick(pi_v, h) + pick(pj_v, 4 + h) for h in range(HEADS)]
            nv2 = [pick(pj_v, 8 + h) for h in range(HEADS)]

            n2 = (alpha[0] * alpha[0] + alpha[1] * alpha[1]) + \
                 (alpha[2] * alpha[2] + alpha[3] * alpha[3])
            n = _sqrt(n2)
            nc = jnp.minimum(jnp.maximum(n, 1e-15), 1.0 - EPS)
            scale = _atanh(nc) / jnp.maximum(n, 1e-15)
            ainf = [alpha[h] * scale for h in range(HEADS)]

            mx = jnp.maximum(jnp.maximum(ainf[0], ainf[1]),
                             jnp.maximum(ainf[2], ainf[3]))
            ex = [jnp.exp(ainf[h] - mx) for h in range(HEADS)]
            rs = 1.0 / ((ex[0] + ex[1]) + (ex[2] + ex[3]))
            sm = [ex[h] * rs for h in range(HEADS)]

            n2p = (sm[0] * sm[0] + sm[1] * sm[1]) + \
                  (sm[2] * sm[2] + sm[3] * sm[3])
            npv = _sqrt(n2p)
            fac = _tanh(npv) / jnp.maximum(npv, 1e-15)

            coef = [None] * HEADS
            for h in range(HEADS):
                nv = _sqrt(nv2[h])
                ncv = jnp.minimum(jnp.maximum(nv, 1e-15), 1.0 - EPS)
                t = (sm[h] * fac) * _atanh(ncv)
                coef[h] = _tanh(t) / jnp.maximum(nv, 1e-15)

            def xjg(f):
                fvec = jnp.full((16,), f, jnp.int32)
                return plsc.load_gather(rows_v, [rows16, fvec])

            out = [coef[0] * xjg(k) for k in range(OUT_F)]
            x2 = nv2[0] * coef[0] * coef[0]
            for h in range(1, HEADS):
                y2 = nv2[h] * coef[h] * coef[h]
                xyp = [out[k] * xjg(32 * h + k) for k in range(4)]
                for k in range(4, OUT_F):
                    xyp[k % 4] = xyp[k % 4] + out[k] * xjg(32 * h + k)
                xy = ((xyp[0] + xyp[1]) + (xyp[2] + xyp[3])) * coef[h]
                cx = 1.0 + 2.0 * xy + y2
                cyb = 1.0 - x2
                cy = cyb * coef[h]
                den = 1.0 + 2.0 * xy + x2 * y2
                rden = 1.0 / jnp.maximum(den, 1e-15)
                cxr = cx * rden
                cyr = cy * rden
                for k in range(OUT_F):
                    out[k] = cxr * out[k] + cyr * xjg(32 * h + k)
                x2 = (cx * cx * x2 + 2.0 * cx * cyb * xy
                      + cyb * cyb * y2) * (rden * rden)

            def _group4(v):
                v1 = v + _lperm(v, perm1)
                return v1 + _lperm(v1, perm2)

            b2 = bpt_v[32, :]
            xyp = [out[k] * bpt_v[k, :] for k in range(4)]
            for k in range(4, OUT_F):
                xyp[k % 4] = xyp[k % 4] + out[k] * bpt_v[k, :]
            xy = _group4((xyp[0] + xyp[1]) + (xyp[2] + xyp[3]))
            x2g = _group4(x2)
            cx = 1.0 + 2.0 * xy + b2
            cy = 1.0 - x2g
            den = 1.0 + 2.0 * xy + x2g * b2
            rden = 1.0 / jnp.maximum(den, 1e-15)
            cxr = cx * rden
            cyr = cy * rden
            rbase = lax.shift_left(rows16, 5)
            for k in range(OUT_F):
                v = cxr * out[k] + cyr * bpt_v[k, :]
                plsc.store_scatter(out_stage, [rbase + k], v)

        obase = pl.multiple_of((ebase0 + c * CHUNK) * OUT_F, 8)
        pltpu.async_copy(out_stage, out_hbm.at[pl.ds(obase, CHUNK * OUT_F)],
                         semo)

    _issue_idx(0, is0, id0, si0)
    _drain_idx(is0, id0, si0)
    _issue_g(is0, id0, rows0, pi0, pj0, sg0)
    _issue_idx(1, is1, id1, si1)

    @pl.loop(0, N_CHUNKS)
    def _chunk(c):
        nxt = jnp.minimum(c + 1, N_CHUNKS - 1)
        nxt2 = jnp.minimum(c + 2, N_CHUNKS - 1)
        even = jnp.bitwise_and(c, 1) == 0

        @pl.when(even)
        def _():
            _drain_g(rows0, pi0, pj0, sg0)
            _issue_idx(nxt2, is0, id0, si0)
            _drain_idx(is1, id1, si1)
            _issue_g(is1, id1, rows1, pi1, pj1, sg1)

            @pl.when(c >= 2)
            def _():
                _drain_out(out0, so0)

            _compute(c, rows0, pi0, pj0, out0, so0)

        @pl.when(jnp.logical_not(even))
        def _():
            _drain_g(rows1, pi1, pj1, sg1)
            _issue_idx(nxt2, is1, id1, si1)
            _drain_idx(is0, id0, si0)
            _issue_g(is0, id0, rows0, pi0, pj0, sg0)

            @pl.when(c >= 2)
            def _():
                _drain_out(out1, so1)

            _compute(c, rows1, pi1, pj1, out1, so1)

    _drain_g(rows1, pi1, pj1, sg1)
    _drain_idx(is0, id0, si0)
    _drain_out(out0, so0)
    _drain_out(out1, so1)


@jax.jit
def kernel(x, edge_index, att, bias):
    def _expmap0(u):
        n = jnp.maximum(jnp.sqrt(jnp.sum(u * u, axis=-1, keepdims=True)), 1e-15)
        return jnp.tanh(n) * u / n

    att_h = _expmap0(att).reshape(HEADS, 2 * OUT_F)
    a1 = att_h[:, :OUT_F].reshape(-1)
    a2 = att_h[:, OUT_F:].reshape(-1)
    abt = jnp.broadcast_to(
        jnp.stack([a1, a2])[:, :, None], (2, D, 16)).astype(jnp.float32)
    bh = _expmap0(bias)
    lanemod = jnp.arange(16) % 4
    b_pat = bh[32 * lanemod[None, :] + jnp.arange(32)[:, None]]
    b2 = jnp.broadcast_to(jnp.sum(bh * bh), (1, 16))
    bpt = jnp.concatenate([b_pat, b2], axis=0).astype(jnp.float32)

    mesh = plsc.VectorSubcoreMesh(core_axis_name="c", subcore_axis_name="s")
    cparams = pltpu.CompilerParams(needs_layout_passes=False,
                                   use_tc_tiling_on_sc=False)

    node_run = pl.kernel(
        _node_body,
        out_type=jax.ShapeDtypeStruct((N_NODES * PCOLS,), jnp.float32),
        mesh=mesh,
        compiler_params=cparams,
        scratch_types=[
            pltpu.VMEM((2, D, 16), jnp.float32),
            pltpu.VMEM((CHUNK, D), jnp.float32),
            pltpu.VMEM((CHUNK * PCOLS,), jnp.float32),
            pltpu.SemaphoreType.DMA,
        ],
    )
    p_tab = node_run(x, abt).reshape(N_NODES, PCOLS)

    edge_run = pl.kernel(
        _edge_body,
        out_type=jax.ShapeDtypeStruct((E_TOTAL * OUT_F,), jnp.float32),
        mesh=mesh,
        compiler_params=cparams,
        scratch_types=[
            pltpu.VMEM((33, 16), jnp.float32),
            pltpu.VMEM((CHUNK,), jnp.int32),
            pltpu.VMEM((CHUNK,), jnp.int32),
            pltpu.VMEM((CHUNK,), jnp.int32),
            pltpu.VMEM((CHUNK,), jnp.int32),
            pltpu.VMEM((CHUNK, D), jnp.float32),
            pltpu.VMEM((CHUNK, PCOLS), jnp.float32),
            pltpu.VMEM((CHUNK, PCOLS), jnp.float32),
            pltpu.VMEM((CHUNK, D), jnp.float32),
            pltpu.VMEM((CHUNK, PCOLS), jnp.float32),
            pltpu.VMEM((CHUNK, PCOLS), jnp.float32),
            pltpu.VMEM((CHUNK * OUT_F,), jnp.float32),
            pltpu.VMEM((CHUNK * OUT_F,), jnp.float32),
            pltpu.SemaphoreType.DMA,
            pltpu.SemaphoreType.DMA,
            pltpu.SemaphoreType.DMA,
            pltpu.SemaphoreType.DMA,
            pltpu.SemaphoreType.DMA,
            pltpu.SemaphoreType.DMA,
        ],
    )
    out = edge_run(x, p_tab, edge_index[0], edge_index[1], bpt)
    return out.reshape(E_TOTAL // 4, D)
